# speedup vs baseline: 14.6131x; 14.6131x over previous
import functools
import jax
import jax.numpy as jnp
from jax.experimental import pallas as pl

_D_PAD = 10240
_DT = 512


def _bind_sum_body(x_ref, id_ref, lvl_ref, q_ref):
    B, F = x_ref.shape
    L = lvl_ref.shape[0]
    idx = jnp.clip(jnp.floor(x_ref[...] * L).astype(jnp.int32), 0, L - 1)
    l_iota = jax.lax.broadcasted_iota(jnp.int32, (B, L, F), 1)
    a = (idx[:, None, :] == l_iota).astype(jnp.bfloat16).reshape(B * L, F)
    p = jax.lax.dot_general(
        a, id_ref[...].astype(jnp.bfloat16),
        (((1,), (0,)), ((), ())),
        preferred_element_type=jnp.float32,
    ).reshape(B, L, -1)
    s = jnp.sum(p * lvl_ref[...][None, :, :], axis=1)
    q_ref[...] = jnp.where(s > 0, 1.0, -1.0).astype(jnp.float32)


def _classify_body(q_ref, cw_ref, out_ref):
    out_ref[...] = jax.lax.dot_general(
        q_ref[...], cw_ref[...],
        (((1,), (1,)), ((), ())),
        preferred_element_type=jnp.float32,
    )


def kernel(x, id_weight, level_weight, classify_weight):
    B, F = x.shape
    L, D = level_weight.shape
    C = classify_weight.shape[0]
    pad = _D_PAD - D
    id_p = jnp.pad(id_weight, ((0, 0), (0, pad)))
    lvl_p = jnp.pad(level_weight, ((0, 0), (0, pad)))
    cw_p = jnp.pad(classify_weight, ((0, 0), (0, pad)))

    grid = _D_PAD // _DT
    q = pl.pallas_call(
        _bind_sum_body,
        grid=(grid,),
        in_specs=[
            pl.BlockSpec((B, F), lambda k: (0, 0)),
            pl.BlockSpec((F, _DT), lambda k: (0, k)),
            pl.BlockSpec((L, _DT), lambda k: (0, k)),
        ],
        out_specs=pl.BlockSpec((B, _DT), lambda k: (0, k)),
        out_shape=jax.ShapeDtypeStruct((B, _D_PAD), jnp.float32),
    )(x, id_p, lvl_p)

    logit = pl.pallas_call(
        _classify_body,
        in_specs=[
            pl.BlockSpec((B, _D_PAD), lambda: (0, 0)),
            pl.BlockSpec((C, _D_PAD), lambda: (0, 0)),
        ],
        out_specs=pl.BlockSpec((B, C), lambda: (0, 0)),
        out_shape=jax.ShapeDtypeStruct((B, C), jnp.float32),
    )(q, cw_p)
    return logit
